# baseline (speedup 1.0000x reference)
import jax
import jax.numpy as jnp
from jax import lax
from jax.experimental import pallas as pl
from jax.experimental.pallas import tpu as pltpu
from jax.experimental.pallas import tpu_sc as plsc

G = 8
NPG = 1250
N = 10000
E = 160000
H = 256
OUT = 128
V = 64
TOPK = 16

NC = 2
NS = 16
LANES = 16

DEG_H = 10240
DEG_PAD = 256
EPT_A = (E + DEG_PAD) // (NC * NS)


def _deg_body(dst_hbm, out_hbm, dst_v, hist_v, acc_v, tmp_v, shared, sem):
    c = lax.axis_index("c")
    s = lax.axis_index("s")
    wid = c * NS + s
    base = wid * EPT_A

    zeros = jnp.zeros((LANES,), jnp.float32)
    ones = jnp.ones((LANES,), jnp.float32)

    def zero_hist(i, _):
        hist_v[pl.ds(i * LANES, LANES)] = zeros
        return 0
    lax.fori_loop(0, DEG_H // LANES, zero_hist, 0)

    pltpu.sync_copy(dst_hbm.at[pl.ds(base, EPT_A)], dst_v)

    def count(i, _):
        idx = dst_v[pl.ds(i * LANES, LANES)]
        plsc.addupdate_scatter(hist_v, [idx], ones)
        return 0
    lax.fori_loop(0, EPT_A // LANES, count, 0)

    pltpu.sync_copy(hist_v, shared.at[s])
    plsc.subcore_barrier()

    cols = DEG_H // NS
    cbase = s * cols
    pltpu.sync_copy(shared.at[0, pl.ds(cbase, cols)], acc_v)
    for slot in range(1, NS):
        pltpu.sync_copy(shared.at[slot, pl.ds(cbase, cols)], tmp_v)

        def addv(i, _):
            acc_v[pl.ds(i * LANES, LANES)] = (
                acc_v[pl.ds(i * LANES, LANES)] + tmp_v[pl.ds(i * LANES, LANES)])
            return 0
        lax.fori_loop(0, cols // LANES, addv, 0)
    pltpu.sync_copy(acc_v, out_hbm.at[c, pl.ds(cbase, cols)])


def _deg_pallas(dst_pad):
    mesh = plsc.VectorSubcoreMesh(core_axis_name="c", subcore_axis_name="s")
    f = pl.kernel(
        _deg_body,
        out_type=jax.ShapeDtypeStruct((NC, DEG_H), jnp.float32),
        mesh=mesh,
        scratch_types=[
            pltpu.VMEM((EPT_A,), jnp.int32),
            pltpu.VMEM((DEG_H,), jnp.float32),
            pltpu.VMEM((DEG_H // NS,), jnp.float32),
            pltpu.VMEM((DEG_H // NS,), jnp.float32),
            pltpu.VMEM_SHARED((NS, DEG_H), jnp.float32),
            pltpu.SemaphoreType.DMA,
        ],
        compiler_params=pltpu.CompilerParams(needs_layout_passes=False),
    )
    return f(dst_pad)


CHUNK = 40
EPT_B = E // NS
HH = H // 2


ZC = 40
NCHUNK = EPT_B // CHUNK
IDXB = 24


def _agg_half(src_hbm, dst_hbm, zeros_hbm, hs_hbm, out_hbm,
              idx_s, idx_d, bufs, agg_sh, gsems, ssems):
    s = lax.axis_index("s")
    zbuf = bufs[0]

    pltpu.sync_copy(zeros_hbm, zbuf)
    for r in range(10):
        pltpu.sync_copy(zbuf, agg_sh.at[pl.ds(s * 640 + r * ZC, ZC)])

    @pl.when(s < NS - 1)
    def _():
        for r in range(10, 16):
            pltpu.sync_copy(zbuf, agg_sh.at[pl.ds(s * 640 + r * ZC, ZC)])
    plsc.subcore_barrier()

    def super_round(q, nb):
        pltpu.sync_copy(src_hbm.at[s, pl.ds(q * IDXB, nb)],
                        idx_s.at[pl.ds(0, nb)])
        pltpu.sync_copy(dst_hbm.at[s, pl.ds(q * IDXB, nb)],
                        idx_d.at[pl.ds(0, nb)])
        seq = [(rb, min(2, nb - rb)) for rb in range(0, nb, 2)]

        def fire_g(i):
            rb, k = seq[i]
            pair = i % 2
            return [pltpu.async_copy(hs_hbm.at[idx_s.at[rb + b]],
                                     bufs[2 * pair + b], gsems[2 * pair + b])
                    for b in range(k)]

        gd = fire_g(0)
        prev_sd = None
        for i in range(len(seq)):
            rb, k = seq[i]
            pair = i % 2
            for d in gd:
                d.wait()
            sd = [pltpu.async_copy(bufs[2 * pair + b],
                                   agg_sh.at[idx_d.at[rb + b]],
                                   ssems[pair], add=True)
                  for b in range(k)]
            if prev_sd is not None:
                for d in prev_sd:
                    d.wait()
            if i + 1 < len(seq):
                gd = fire_g(i + 1)
            prev_sd = sd
        for d in prev_sd:
            d.wait()

    def full_sr(q, _):
        super_round(q, IDXB)
        return 0
    lax.fori_loop(0, NCHUNK // IDXB, full_sr, 0)
    if NCHUNK % IDXB:
        super_round(NCHUNK // IDXB, NCHUNK % IDXB)
    plsc.subcore_barrier()

    def copy_out(r):
        off = pl.multiple_of(s * 640 + r * ZC, 8)
        pltpu.sync_copy(agg_sh.at[pl.ds(off, ZC)], zbuf)
        pltpu.sync_copy(zbuf, out_hbm.at[pl.ds(off, ZC)])

    for r in range(10):
        copy_out(r)

    @pl.when(s < NS - 1)
    def _():
        for r in range(10, 16):
            copy_out(r)


def _agg_body(src_hbm, dst_hbm, zeros_hbm, hs_lo, hs_hi, out_lo, out_hi,
              idx_s, idx_d, b0, b1, b2, b3, agg_sh,
              g0, g1, g2, g3, ss0, ss1):
    c = lax.axis_index("c")
    bufs = [b0, b1, b2, b3]
    gsems = [g0, g1, g2, g3]
    ssems = [ss0, ss1]

    @pl.when(c == 0)
    def _():
        _agg_half(src_hbm, dst_hbm, zeros_hbm, hs_lo, out_lo,
                  idx_s, idx_d, bufs, agg_sh, gsems, ssems)

    @pl.when(c == 1)
    def _():
        _agg_half(src_hbm, dst_hbm, zeros_hbm, hs_hi, out_hi,
                  idx_s, idx_d, bufs, agg_sh, gsems, ssems)


def _agg_pallas(src, dst, hs_lo, hs_hi):
    mesh = plsc.VectorSubcoreMesh(core_axis_name="c", subcore_axis_name="s")
    f = pl.kernel(
        _agg_body,
        out_type=(jax.ShapeDtypeStruct((N, HH), jnp.float32),
                  jax.ShapeDtypeStruct((N, HH), jnp.float32)),
        mesh=mesh,
        scratch_types=(
            [pltpu.VMEM((IDXB, CHUNK), jnp.int32),
             pltpu.VMEM((IDXB, CHUNK), jnp.int32)]
            + [pltpu.VMEM((CHUNK, HH), jnp.float32) for _ in range(4)]
            + [pltpu.VMEM_SHARED((N, HH), jnp.float32)]
            + [pltpu.SemaphoreType.DMA for _ in range(6)]
        ),
        compiler_params=pltpu.CompilerParams(needs_layout_passes=False),
    )
    zeros = jnp.zeros((ZC, HH), jnp.float32)
    src3 = src.reshape(NS, NCHUNK, CHUNK)
    dst3 = dst.reshape(NS, NCHUNK, CHUNK)
    return f(src3, dst3, zeros, hs_lo, hs_hi)


RB = 1000


def _emb_kernel(x_ref, we_ref, be_ref, wg_ref, dega_ref, degb_ref,
                lo_ref, hi_ref, wc_ref, bc_ref):
    @pl.when(pl.program_id(0) == 0)
    def _():
        wc_ref[...] = jnp.dot(we_ref[...], wg_ref[...],
                              preferred_element_type=jnp.float32)
        bc_ref[...] = jnp.dot(be_ref[...], wg_ref[...],
                              preferred_element_type=jnp.float32)

    h2 = jnp.dot(x_ref[...], wc_ref[...],
                 preferred_element_type=jnp.float32) + bc_ref[...]
    deg = dega_ref[...] + degb_ref[...] + 1.0
    dinv = lax.rsqrt(jnp.maximum(deg, 1.0))
    hs = h2 * dinv
    lo_ref[...] = hs[:, :HH]
    hi_ref[...] = hs[:, HH:]


def _emb_pallas(x, W_emb, b_emb, W_gcn, dega, degb):
    grid = (N // RB,)
    return pl.pallas_call(
        _emb_kernel,
        grid=grid,
        in_specs=[
            pl.BlockSpec((RB, H), lambda i: (i, 0)),
            pl.BlockSpec((H, H), lambda i: (0, 0)),
            pl.BlockSpec((1, H), lambda i: (0, 0)),
            pl.BlockSpec((H, H), lambda i: (0, 0)),
            pl.BlockSpec((RB, 1), lambda i: (i, 0)),
            pl.BlockSpec((RB, 1), lambda i: (i, 0)),
        ],
        out_specs=[
            pl.BlockSpec((RB, HH), lambda i: (i, 0)),
            pl.BlockSpec((RB, HH), lambda i: (i, 0)),
        ],
        out_shape=[
            jax.ShapeDtypeStruct((N, HH), jnp.float32),
            jax.ShapeDtypeStruct((N, HH), jnp.float32),
        ],
        scratch_shapes=[
            pltpu.VMEM((H, H), jnp.float32),
            pltpu.VMEM((1, H), jnp.float32),
        ],
    )(x, W_emb, b_emb, W_gcn, dega, degb)


def _tail_kernel(hs_lo, hs_hi, agg_lo, agg_hi, dega, degb, bg_ref,
                 aw1, ab1, aw2, ab2, vnet, ew_in,
                 vw1, vb1, vw2, vb2, ow1, ob1, ow2, ob2, out_ref):
    deg = dega[0] + degb[0] + 1.0
    dinv = lax.rsqrt(jnp.maximum(deg, 1.0))
    glo = jnp.maximum(dinv * (agg_lo[0] + hs_lo[0]) + bg_ref[:, :HH], 0.0)
    ghi = jnp.maximum(dinv * (agg_hi[0] + hs_hi[0]) + bg_ref[:, HH:], 0.0)
    gx = jnp.concatenate([glo, ghi], axis=1)

    a1 = jnp.maximum(jnp.dot(gx, aw1[...],
                             preferred_element_type=jnp.float32) + ab1[...], 0.0)
    aff = jnp.dot(a1, aw2[...], preferred_element_type=jnp.float32) + ab2[...]
    att = lax.dot_general(vnet[0], aff, (((1,), (1,)), ((), ())),
                          preferred_element_type=jnp.float32)
    ew = ew_in[0] * (1.0 + att)

    work = ew
    for _ in range(TOPK - 1):
        m15 = jnp.max(work, axis=0, keepdims=True)
        work = jnp.where(work == m15, -jnp.inf, work)
    t16 = jnp.max(work, axis=0, keepdims=True)
    ewm = jnp.where(ew >= t16, ew, 0.0)

    m = jnp.max(ewm, axis=0, keepdims=True)
    ex = jnp.exp(ewm - m)
    sm = ex / jnp.sum(ex, axis=0, keepdims=True)

    vn = jnp.dot(sm, gx, preferred_element_type=jnp.float32)
    v1 = jnp.maximum(jnp.dot(vn, vw1[...],
                             preferred_element_type=jnp.float32) + vb1[...], 0.0)
    v2 = jnp.dot(v1, vw2[...], preferred_element_type=jnp.float32) + vb2[...]
    gf = jnp.mean(v2, axis=0, keepdims=True)
    o1 = jnp.maximum(jnp.dot(gf, ow1[...],
                             preferred_element_type=jnp.float32) + ob1[...], 0.0)
    out_ref[0] = jnp.dot(o1, ow2[...],
                         preferred_element_type=jnp.float32) + ob2[...]


def _tail_pallas(hs_lo, hs_hi, agg_lo, agg_hi, dega, degb, b_gcn,
                 aff_W1, aff_b1, aff_W2, aff_b2, vne_t, edge_weights,
                 vn_W1, vn_b1, vn_W2, vn_b2, out_W1, out_b1, out_W2, out_b2):
    grid = (G,)
    row3 = lambda i: (i, 0, 0)
    full = lambda i: (0, 0)
    return pl.pallas_call(
        _tail_kernel,
        grid=grid,
        in_specs=[
            pl.BlockSpec((1, NPG, HH), row3),
            pl.BlockSpec((1, NPG, HH), row3),
            pl.BlockSpec((1, NPG, HH), row3),
            pl.BlockSpec((1, NPG, HH), row3),
            pl.BlockSpec((1, NPG, 1), row3),
            pl.BlockSpec((1, NPG, 1), row3),
            pl.BlockSpec((1, H), full),
            pl.BlockSpec((H, H), full),
            pl.BlockSpec((1, H), full),
            pl.BlockSpec((H, H), full),
            pl.BlockSpec((1, H), full),
            pl.BlockSpec((1, V, H), lambda i: (i, 0, 0)),
            pl.BlockSpec((1, V, NPG), lambda i: (i, 0, 0)),
            pl.BlockSpec((H, H), full),
            pl.BlockSpec((1, H), full),
            pl.BlockSpec((H, H), full),
            pl.BlockSpec((1, H), full),
            pl.BlockSpec((H, H), full),
            pl.BlockSpec((1, H), full),
            pl.BlockSpec((H, OUT), full),
            pl.BlockSpec((1, OUT), full),
        ],
        out_specs=pl.BlockSpec((1, 1, OUT), row3),
        out_shape=jax.ShapeDtypeStruct((G, 1, OUT), jnp.float32),
    )(hs_lo, hs_hi, agg_lo, agg_hi, dega, degb, b_gcn,
      aff_W1, aff_b1, aff_W2, aff_b2, vne_t, edge_weights,
      vn_W1, vn_b1, vn_W2, vn_b2, out_W1, out_b1, out_W2, out_b2)


def kernel(x, edge_index, batch, W_emb, b_emb, W_gcn, b_gcn,
           aff_W1, aff_b1, aff_W2, aff_b2, vn_W1, vn_b1, vn_W2, vn_b2,
           out_W1, out_b1, out_W2, out_b2, edge_weights, vne):
    src = edge_index[0].astype(jnp.int32)
    dst = edge_index[1].astype(jnp.int32)
    dst_pad = jnp.concatenate(
        [dst, jnp.full((DEG_PAD,), N, jnp.int32)])

    deg_p = _deg_pallas(dst_pad)
    dega = deg_p[0, :N].reshape(N, 1)
    degb = deg_p[1, :N].reshape(N, 1)

    hs_lo, hs_hi = _emb_pallas(
        x, W_emb, b_emb.reshape(1, H), W_gcn, dega, degb)

    agg_lo, agg_hi = _agg_pallas(src, dst, hs_lo, hs_hi)

    g3 = lambda a: a.reshape(G, NPG, -1)
    out = _tail_pallas(
        g3(hs_lo), g3(hs_hi), g3(agg_lo), g3(agg_hi), g3(dega), g3(degb),
        b_gcn.reshape(1, H),
        aff_W1, aff_b1.reshape(1, H), aff_W2, aff_b2.reshape(1, H),
        vne, edge_weights.transpose(0, 2, 1),
        vn_W1, vn_b1.reshape(1, H), vn_W2, vn_b2.reshape(1, H),
        out_W1, out_b1.reshape(1, H), out_W2, out_b2.reshape(1, OUT))
    return out.reshape(G, OUT)


# speedup vs baseline: 1.0222x; 1.0222x over previous
import jax
import jax.numpy as jnp
from jax import lax
from jax.experimental import pallas as pl
from jax.experimental.pallas import tpu as pltpu
from jax.experimental.pallas import tpu_sc as plsc

G = 8
NPG = 1250
N = 10000
E = 160000
H = 256
OUT = 128
V = 64
TOPK = 16

NC = 2
NS = 16
LANES = 16

DEG_H = 10240
DEG_PAD = 256
EPT_A = (E + DEG_PAD) // (NC * NS)


def _deg_body(dst_hbm, out_hbm, dst_v, hist_v, acc_v, tmp_v, shared, sem):
    c = lax.axis_index("c")
    s = lax.axis_index("s")
    wid = c * NS + s
    base = wid * EPT_A

    zeros = jnp.zeros((LANES,), jnp.float32)
    ones = jnp.ones((LANES,), jnp.float32)

    def zero_hist(i, _):
        hist_v[pl.ds(i * LANES, LANES)] = zeros
        return 0
    lax.fori_loop(0, DEG_H // LANES, zero_hist, 0)

    pltpu.sync_copy(dst_hbm.at[pl.ds(base, EPT_A)], dst_v)

    def count(i, _):
        idx = dst_v[pl.ds(i * LANES, LANES)]
        plsc.addupdate_scatter(hist_v, [idx], ones)
        return 0
    lax.fori_loop(0, EPT_A // LANES, count, 0)

    pltpu.sync_copy(hist_v, shared.at[s])
    plsc.subcore_barrier()

    cols = DEG_H // NS
    cbase = s * cols
    pltpu.sync_copy(shared.at[0, pl.ds(cbase, cols)], acc_v)
    for slot in range(1, NS):
        pltpu.sync_copy(shared.at[slot, pl.ds(cbase, cols)], tmp_v)

        def addv(i, _):
            acc_v[pl.ds(i * LANES, LANES)] = (
                acc_v[pl.ds(i * LANES, LANES)] + tmp_v[pl.ds(i * LANES, LANES)])
            return 0
        lax.fori_loop(0, cols // LANES, addv, 0)
    pltpu.sync_copy(acc_v, out_hbm.at[c, pl.ds(cbase, cols)])


def _deg_pallas(dst_pad):
    mesh = plsc.VectorSubcoreMesh(core_axis_name="c", subcore_axis_name="s")
    f = pl.kernel(
        _deg_body,
        out_type=jax.ShapeDtypeStruct((NC, DEG_H), jnp.float32),
        mesh=mesh,
        scratch_types=[
            pltpu.VMEM((EPT_A,), jnp.int32),
            pltpu.VMEM((DEG_H,), jnp.float32),
            pltpu.VMEM((DEG_H // NS,), jnp.float32),
            pltpu.VMEM((DEG_H // NS,), jnp.float32),
            pltpu.VMEM_SHARED((NS, DEG_H), jnp.float32),
            pltpu.SemaphoreType.DMA,
        ],
        compiler_params=pltpu.CompilerParams(needs_layout_passes=False),
    )
    return f(dst_pad)


CHUNK = 40
EPT_B = E // NS
HH = H // 2


ZC = 40
NCHUNK = EPT_B // CHUNK
IDXB = 40


def _agg_half(src_hbm, dst_hbm, zeros_hbm, hs_hbm, out_hbm,
              idx_s, idx_d, bufs, agg_sh, gsems, ssems):
    s = lax.axis_index("s")
    zbuf = bufs[0]

    pltpu.sync_copy(zeros_hbm, zbuf)
    for r in range(10):
        pltpu.sync_copy(zbuf, agg_sh.at[pl.ds(s * 640 + r * ZC, ZC)])

    @pl.when(s < NS - 1)
    def _():
        for r in range(10, 16):
            pltpu.sync_copy(zbuf, agg_sh.at[pl.ds(s * 640 + r * ZC, ZC)])
    plsc.subcore_barrier()

    def super_round(q, nb):
        pltpu.sync_copy(src_hbm.at[s, pl.ds(q * IDXB, nb)],
                        idx_s.at[pl.ds(0, nb)])
        pltpu.sync_copy(dst_hbm.at[s, pl.ds(q * IDXB, nb)],
                        idx_d.at[pl.ds(0, nb)])
        seq = [(rb, min(2, nb - rb)) for rb in range(0, nb, 2)]

        def fire_g(i):
            rb, k = seq[i]
            pair = i % 2
            return [pltpu.async_copy(hs_hbm.at[idx_s.at[rb + b]],
                                     bufs[2 * pair + b], gsems[2 * pair + b])
                    for b in range(k)]

        gd = fire_g(0)
        prev_sd = None
        for i in range(len(seq)):
            rb, k = seq[i]
            pair = i % 2
            for d in gd:
                d.wait()
            sd = [pltpu.async_copy(bufs[2 * pair + b],
                                   agg_sh.at[idx_d.at[rb + b]],
                                   ssems[pair], add=True)
                  for b in range(k)]
            if prev_sd is not None:
                for d in prev_sd:
                    d.wait()
            if i + 1 < len(seq):
                gd = fire_g(i + 1)
            prev_sd = sd
        for d in prev_sd:
            d.wait()

    def full_sr(q, _):
        super_round(q, IDXB)
        return 0
    lax.fori_loop(0, NCHUNK // IDXB, full_sr, 0)
    if NCHUNK % IDXB:
        super_round(NCHUNK // IDXB, NCHUNK % IDXB)
    plsc.subcore_barrier()

    def copy_out(r):
        off = pl.multiple_of(s * 640 + r * ZC, 8)
        pltpu.sync_copy(agg_sh.at[pl.ds(off, ZC)], zbuf)
        pltpu.sync_copy(zbuf, out_hbm.at[pl.ds(off, ZC)])

    for r in range(10):
        copy_out(r)

    @pl.when(s < NS - 1)
    def _():
        for r in range(10, 16):
            copy_out(r)


def _agg_body(src_hbm, dst_hbm, zeros_hbm, hs_lo, hs_hi, out_lo, out_hi,
              idx_s, idx_d, b0, b1, b2, b3, agg_sh,
              g0, g1, g2, g3, ss0, ss1):
    c = lax.axis_index("c")
    bufs = [b0, b1, b2, b3]
    gsems = [g0, g1, g2, g3]
    ssems = [ss0, ss1]

    @pl.when(c == 0)
    def _():
        _agg_half(src_hbm, dst_hbm, zeros_hbm, hs_lo, out_lo,
                  idx_s, idx_d, bufs, agg_sh, gsems, ssems)

    @pl.when(c == 1)
    def _():
        _agg_half(src_hbm, dst_hbm, zeros_hbm, hs_hi, out_hi,
                  idx_s, idx_d, bufs, agg_sh, gsems, ssems)


def _agg_pallas(src, dst, hs_lo, hs_hi):
    mesh = plsc.VectorSubcoreMesh(core_axis_name="c", subcore_axis_name="s")
    f = pl.kernel(
        _agg_body,
        out_type=(jax.ShapeDtypeStruct((N, HH), jnp.float32),
                  jax.ShapeDtypeStruct((N, HH), jnp.float32)),
        mesh=mesh,
        scratch_types=(
            [pltpu.VMEM((IDXB, CHUNK), jnp.int32),
             pltpu.VMEM((IDXB, CHUNK), jnp.int32)]
            + [pltpu.VMEM((CHUNK, HH), jnp.float32) for _ in range(4)]
            + [pltpu.VMEM_SHARED((N, HH), jnp.float32)]
            + [pltpu.SemaphoreType.DMA for _ in range(6)]
        ),
        compiler_params=pltpu.CompilerParams(needs_layout_passes=False),
    )
    zeros = jnp.zeros((ZC, HH), jnp.float32)
    src3 = src.reshape(NS, NCHUNK, CHUNK)
    dst3 = dst.reshape(NS, NCHUNK, CHUNK)
    return f(src3, dst3, zeros, hs_lo, hs_hi)


RB = 1000


def _emb_kernel(x_ref, we_ref, be_ref, wg_ref, dega_ref, degb_ref,
                lo_ref, hi_ref, wc_ref, bc_ref):
    @pl.when(pl.program_id(0) == 0)
    def _():
        wc_ref[...] = jnp.dot(we_ref[...], wg_ref[...],
                              preferred_element_type=jnp.float32)
        bc_ref[...] = jnp.dot(be_ref[...], wg_ref[...],
                              preferred_element_type=jnp.float32)

    h2 = jnp.dot(x_ref[...], wc_ref[...],
                 preferred_element_type=jnp.float32) + bc_ref[...]
    deg = dega_ref[...] + degb_ref[...] + 1.0
    dinv = lax.rsqrt(jnp.maximum(deg, 1.0))
    hs = h2 * dinv
    lo_ref[...] = hs[:, :HH]
    hi_ref[...] = hs[:, HH:]


def _emb_pallas(x, W_emb, b_emb, W_gcn, dega, degb):
    grid = (N // RB,)
    return pl.pallas_call(
        _emb_kernel,
        grid=grid,
        in_specs=[
            pl.BlockSpec((RB, H), lambda i: (i, 0)),
            pl.BlockSpec((H, H), lambda i: (0, 0)),
            pl.BlockSpec((1, H), lambda i: (0, 0)),
            pl.BlockSpec((H, H), lambda i: (0, 0)),
            pl.BlockSpec((RB, 1), lambda i: (i, 0)),
            pl.BlockSpec((RB, 1), lambda i: (i, 0)),
        ],
        out_specs=[
            pl.BlockSpec((RB, HH), lambda i: (i, 0)),
            pl.BlockSpec((RB, HH), lambda i: (i, 0)),
        ],
        out_shape=[
            jax.ShapeDtypeStruct((N, HH), jnp.float32),
            jax.ShapeDtypeStruct((N, HH), jnp.float32),
        ],
        scratch_shapes=[
            pltpu.VMEM((H, H), jnp.float32),
            pltpu.VMEM((1, H), jnp.float32),
        ],
    )(x, W_emb, b_emb, W_gcn, dega, degb)


def _tail_kernel(hs_lo, hs_hi, agg_lo, agg_hi, dega, degb, bg_ref,
                 aw1, ab1, aw2, ab2, vnet, ew_in,
                 vw1, vb1, vw2, vb2, ow1, ob1, ow2, ob2, out_ref):
    deg = dega[0] + degb[0] + 1.0
    dinv = lax.rsqrt(jnp.maximum(deg, 1.0))
    glo = jnp.maximum(dinv * (agg_lo[0] + hs_lo[0]) + bg_ref[:, :HH], 0.0)
    ghi = jnp.maximum(dinv * (agg_hi[0] + hs_hi[0]) + bg_ref[:, HH:], 0.0)
    gx = jnp.concatenate([glo, ghi], axis=1)

    a1 = jnp.maximum(jnp.dot(gx, aw1[...],
                             preferred_element_type=jnp.float32) + ab1[...], 0.0)
    aff = jnp.dot(a1, aw2[...], preferred_element_type=jnp.float32) + ab2[...]
    att = lax.dot_general(vnet[0], aff, (((1,), (1,)), ((), ())),
                          preferred_element_type=jnp.float32)
    ew = ew_in[0] * (1.0 + att)

    work = ew
    for _ in range(TOPK - 1):
        m15 = jnp.max(work, axis=0, keepdims=True)
        work = jnp.where(work == m15, -jnp.inf, work)
    t16 = jnp.max(work, axis=0, keepdims=True)
    ewm = jnp.where(ew >= t16, ew, 0.0)

    m = jnp.max(ewm, axis=0, keepdims=True)
    ex = jnp.exp(ewm - m)
    sm = ex / jnp.sum(ex, axis=0, keepdims=True)

    vn = jnp.dot(sm, gx, preferred_element_type=jnp.float32)
    v1 = jnp.maximum(jnp.dot(vn, vw1[...],
                             preferred_element_type=jnp.float32) + vb1[...], 0.0)
    v2 = jnp.dot(v1, vw2[...], preferred_element_type=jnp.float32) + vb2[...]
    gf = jnp.mean(v2, axis=0, keepdims=True)
    o1 = jnp.maximum(jnp.dot(gf, ow1[...],
                             preferred_element_type=jnp.float32) + ob1[...], 0.0)
    out_ref[0] = jnp.dot(o1, ow2[...],
                         preferred_element_type=jnp.float32) + ob2[...]


def _tail_pallas(hs_lo, hs_hi, agg_lo, agg_hi, dega, degb, b_gcn,
                 aff_W1, aff_b1, aff_W2, aff_b2, vne_t, edge_weights,
                 vn_W1, vn_b1, vn_W2, vn_b2, out_W1, out_b1, out_W2, out_b2):
    grid = (G,)
    row3 = lambda i: (i, 0, 0)
    full = lambda i: (0, 0)
    return pl.pallas_call(
        _tail_kernel,
        grid=grid,
        in_specs=[
            pl.BlockSpec((1, NPG, HH), row3),
            pl.BlockSpec((1, NPG, HH), row3),
            pl.BlockSpec((1, NPG, HH), row3),
            pl.BlockSpec((1, NPG, HH), row3),
            pl.BlockSpec((1, NPG, 1), row3),
            pl.BlockSpec((1, NPG, 1), row3),
            pl.BlockSpec((1, H), full),
            pl.BlockSpec((H, H), full),
            pl.BlockSpec((1, H), full),
            pl.BlockSpec((H, H), full),
            pl.BlockSpec((1, H), full),
            pl.BlockSpec((1, V, H), lambda i: (i, 0, 0)),
            pl.BlockSpec((1, V, NPG), lambda i: (i, 0, 0)),
            pl.BlockSpec((H, H), full),
            pl.BlockSpec((1, H), full),
            pl.BlockSpec((H, H), full),
            pl.BlockSpec((1, H), full),
            pl.BlockSpec((H, H), full),
            pl.BlockSpec((1, H), full),
            pl.BlockSpec((H, OUT), full),
            pl.BlockSpec((1, OUT), full),
        ],
        out_specs=pl.BlockSpec((1, 1, OUT), row3),
        out_shape=jax.ShapeDtypeStruct((G, 1, OUT), jnp.float32),
    )(hs_lo, hs_hi, agg_lo, agg_hi, dega, degb, b_gcn,
      aff_W1, aff_b1, aff_W2, aff_b2, vne_t, edge_weights,
      vn_W1, vn_b1, vn_W2, vn_b2, out_W1, out_b1, out_W2, out_b2)


def kernel(x, edge_index, batch, W_emb, b_emb, W_gcn, b_gcn,
           aff_W1, aff_b1, aff_W2, aff_b2, vn_W1, vn_b1, vn_W2, vn_b2,
           out_W1, out_b1, out_W2, out_b2, edge_weights, vne):
    src = edge_index[0].astype(jnp.int32)
    dst = edge_index[1].astype(jnp.int32)
    dst_pad = jnp.concatenate(
        [dst, jnp.full((DEG_PAD,), N, jnp.int32)])

    deg_p = _deg_pallas(dst_pad)
    dega = deg_p[0, :N].reshape(N, 1)
    degb = deg_p[1, :N].reshape(N, 1)

    hs_lo, hs_hi = _emb_pallas(
        x, W_emb, b_emb.reshape(1, H), W_gcn, dega, degb)

    agg_lo, agg_hi = _agg_pallas(src, dst, hs_lo, hs_hi)

    g3 = lambda a: a.reshape(G, NPG, -1)
    out = _tail_pallas(
        g3(hs_lo), g3(hs_hi), g3(agg_lo), g3(agg_hi), g3(dega), g3(degb),
        b_gcn.reshape(1, H),
        aff_W1, aff_b1.reshape(1, H), aff_W2, aff_b2.reshape(1, H),
        vne, edge_weights.transpose(0, 2, 1),
        vn_W1, vn_b1.reshape(1, H), vn_W2, vn_b2.reshape(1, H),
        out_W1, out_b1.reshape(1, H), out_W2, out_b2.reshape(1, OUT))
    return out.reshape(G, OUT)
